# baseline (speedup 1.0000x reference)
import functools

import jax
import jax.numpy as jnp
from jax import lax
from jax.experimental import pallas as pl
from jax.experimental.pallas import tpu as pltpu
from jax.experimental.pallas import tpu_sc as plsc

_B = 16384
_E = 16
_L = 16
_NC = 2
_NT = 16
_RPT = _B // _NT
_OPW = _B // (_NC * _NT)
_CH = 128
_NCH = _RPT // _CH
_OCH = _OPW // _CH


def _fm_body(uidx_hbm, iidx_hbm, uemb_hbm, ubias_hbm, iemb_hbm, ibias_hbm,
             out_hbm,
             uidx_v, iidx_v, urows_v, irows_v, ub_v, ib_v,
             out_v, pacc_v, zidx_v, zero_v, bcast_v, shared_acc, sem):
    c = lax.axis_index("c")
    s = lax.axis_index("s")
    base = s * _RPT

    zidx_v[...] = jnp.zeros((_L,), jnp.int32)

    @pl.when(s == 0)
    def _zero_shared():
        zero_v[...] = jnp.zeros((_L,), jnp.float32)
        pltpu.sync_copy(zero_v, shared_acc)
    plsc.subcore_barrier()

    pltpu.sync_copy(uidx_hbm.at[pl.ds(s * _NCH, _NCH)], uidx_v)
    pltpu.sync_copy(iidx_hbm.at[pl.ds(s * _NCH, _NCH)], iidx_v)

    copies = []
    for k in range(_NCH):
        copies.append(pltpu.async_copy(
            uemb_hbm.at[uidx_v.at[k]], urows_v.at[pl.ds(k * _CH, _CH)], sem))
        copies.append(pltpu.async_copy(
            iemb_hbm.at[iidx_v.at[k]], irows_v.at[pl.ds(k * _CH, _CH)], sem))
    for k in range(_OCH):
        copies.append(pltpu.async_copy(
            ubias_hbm.at[uidx_v.at[c * _OCH + k]], ub_v.at[k], sem))
        copies.append(pltpu.async_copy(
            ibias_hbm.at[iidx_v.at[c * _OCH + k]], ib_v.at[k], sem))
    for cp in copies:
        cp.wait()

    def dot_step(r, acc):
        return acc + urows_v[r] * irows_v[r]
    acc = lax.fori_loop(0, _RPT, dot_step, jnp.zeros((_L,), jnp.float32),
                        unroll=8)
    pacc_v[...] = acc

    pltpu.sync_copy(pacc_v, shared_acc.at[zidx_v], add=True)
    plsc.subcore_barrier()
    pltpu.async_copy(shared_acc.at[zidx_v], bcast_v, sem).wait()
    s_vec = bcast_v[...]

    for k in range(_OCH):
        for j in range(_CH // _L):
            ub16 = ub_v[k, pl.ds(j * _L, _L)]
            ib16 = ib_v[k, pl.ds(j * _L, _L)]
            out_v[pl.ds((k * _CH + j * _L), _L)] = ub16 + ib16 + s_vec
    pltpu.sync_copy(out_v, out_hbm.at[pl.ds(base + c * _OPW, _OPW)])


@jax.jit
def _fm_call(uidx, iidx, uemb, ubias, iemb, ibias):
    mesh = plsc.VectorSubcoreMesh(core_axis_name="c", subcore_axis_name="s")
    kern = functools.partial(
        pl.kernel,
        out_type=jax.ShapeDtypeStruct((_B,), jnp.float32),
        mesh=mesh,
        compiler_params=pltpu.CompilerParams(use_tc_tiling_on_sc=False),
        scratch_types=[
            pltpu.VMEM((_NCH, _CH), jnp.int32),
            pltpu.VMEM((_NCH, _CH), jnp.int32),
            pltpu.VMEM((_RPT, _E), jnp.float32),
            pltpu.VMEM((_RPT, _E), jnp.float32),
            pltpu.VMEM((_OCH, _CH), jnp.float32),
            pltpu.VMEM((_OCH, _CH), jnp.float32),
            pltpu.VMEM((_OPW,), jnp.float32),
            pltpu.VMEM((_L,), jnp.float32),
            pltpu.VMEM((_L,), jnp.int32),
            pltpu.VMEM((_L,), jnp.float32),
            pltpu.VMEM((_L,), jnp.float32),
            pltpu.VMEM_SHARED((_L,), jnp.float32),
            pltpu.SemaphoreType.DMA,
        ],
    )(_fm_body)
    return kern(uidx, iidx, uemb, ubias, iemb, ibias)


def kernel(inputs, user_emb, user_bias, item_emb, item_bias):
    u_idx = inputs[:, 0].reshape(_B // _CH, _CH)
    i_idx = inputs[:, 1].reshape(_B // _CH, _CH)
    ubias = user_bias.reshape(-1)
    ibias = item_bias.reshape(-1)
    out = _fm_call(u_idx, i_idx, user_emb, ubias, item_emb, ibias)
    return out.reshape(_B, 1)
